# baseline (speedup 1.0000x reference)
import functools

import jax
import jax.numpy as jnp
from jax import lax
from jax.experimental import pallas as pl
from jax.experimental.pallas import tpu as pltpu
from jax.experimental.pallas import tpu_sc as plsc

B, S, H, V = 4, 8192, 128, 100
BS = B * S
VP = 128
CHUNK = 128
NBUF = 4
TBLK = 512


def _tables_body(emb_ref, w1t_ref, b1_ref, w2t_ref, b2_ref, t12_ref):
    t1 = jnp.dot(emb_ref[:], w1t_ref[:],
                 preferred_element_type=jnp.float32) + b1_ref[:]
    t12_ref[:, :H] = t1
    t12_ref[:, H:] = jnp.dot(t1, w2t_ref[:],
                             preferred_element_type=jnp.float32) + b2_ref[:]


def _compute_tables(emb_pad, w1t, b1r, w2t, b2r):
    return pl.pallas_call(
        _tables_body,
        out_shape=jax.ShapeDtypeStruct((VP, 2 * H), jnp.float32),
    )(emb_pad, w1t, b1r, w2t, b2r)


def _onehot_body(ids_ref, t12_ref, y1_ref, y2_ref):
    ids = ids_ref[0, 0]
    oh = (ids.reshape(TBLK, 1)
          == lax.broadcasted_iota(jnp.int32, (TBLK, VP), 1))
    y12 = jnp.dot(oh.astype(jnp.float32), t12_ref[:],
                  preferred_element_type=jnp.float32)
    y1_ref[:] = y12[:, :H]
    y2_ref[:] = y12[:, H:]


def _tc_lookup(ids3, t12):
    nblk = BS // TBLK
    return pl.pallas_call(
        _onehot_body,
        grid=(nblk,),
        in_specs=[
            pl.BlockSpec((1, 1, TBLK), lambda i: (i, 0, 0)),
            pl.BlockSpec((VP, 2 * H), lambda i: (0, 0)),
        ],
        out_specs=[
            pl.BlockSpec((TBLK, H), lambda i: (i, 0)),
            pl.BlockSpec((TBLK, H), lambda i: (i, 0)),
        ],
        out_shape=[jax.ShapeDtypeStruct((BS, H), jnp.float32)] * 2,
    )(ids3, t12)


def _make_sc_gather(ns):
    tok_per_w = BS // ns
    nchunk = tok_per_w // CHUNK
    mesh = plsc.VectorSubcoreMesh(core_axis_name="c", subcore_axis_name="s",
                                  num_cores=1)

    @functools.partial(
        pl.kernel,
        out_type=jax.ShapeDtypeStruct((BS, H), jnp.float32),
        mesh=mesh,
        scratch_types=[
            pltpu.VMEM((nchunk, CHUNK), jnp.int32),
            pltpu.VMEM((NBUF, CHUNK, H), jnp.float32),
            pltpu.VMEM_SHARED((VP, H), jnp.float32),
            pltpu.SemaphoreType.DMA,
            pltpu.SemaphoreType.DMA,
        ],
    )
    def sc_gather(ids_hbm, t0, y0, idx_v, buf, tv0, gsem, wsem):
        wid = lax.axis_index("s")
        base = wid * tok_per_w
        pltpu.sync_copy(ids_hbm.at[pl.ds(wid * nchunk, nchunk)], idx_v)

        @pl.when(wid == 0)
        def _stage_table():
            pltpu.sync_copy(t0, tv0)

        plsc.subcore_barrier()

        def gather(i):
            return pltpu.async_copy(tv0.at[idx_v.at[i]], buf.at[i % NBUF],
                                    gsem)

        def write(i):
            return pltpu.async_copy(buf.at[i % NBUF],
                                    y0.at[pl.ds(base + i * CHUNK, CHUNK)],
                                    wsem)

        gds = [gather(i) for i in range(NBUF - 1)]
        wds = []
        for i in range(nchunk):
            if i + NBUF - 1 < nchunk:
                if i >= 1:
                    wds[i - 1].wait()
                gds.append(gather(i + NBUF - 1))
            elif i >= 1:
                wds[i - 1].wait()
            gds[i].wait()
            wds.append(write(i))
        wds[-1].wait()

    return sc_gather


def kernel(input_ids, emb, W1, b1, W2, b2):
    info = plsc.get_sparse_core_info()
    emb_pad = jnp.zeros((VP, H), jnp.float32).at[:V].set(emb)
    t12 = _compute_tables(emb_pad, W1.T, b1.reshape(1, H),
                          W2.T, b2.reshape(1, H))
    ids2 = input_ids.reshape(BS // CHUNK, CHUNK)
    ids3 = input_ids.reshape(BS // TBLK, 1, TBLK)
    sc_gather = _make_sc_gather(info.num_subcores)
    y0 = sc_gather(ids2, emb_pad)
    y1, y2 = _tc_lookup(ids3, t12)
    return (y0.reshape(B, S, H), y1.reshape(B, S, H), y2.reshape(B, S, H))
